# baseline (speedup 1.0000x reference)
import functools

import jax
import jax.numpy as jnp
from jax import lax
from jax.experimental import pallas as pl
from jax.experimental.pallas import tpu as pltpu
from jax.experimental.pallas import tpu_sc as plsc

N = 50000
NBR = 16

NC = 2
NS = 16
NW = NC * NS

BLK = 80
NBLKS = N // BLK

RB = 2000


@functools.partial(
    pl.kernel,
    out_type=[
        jax.ShapeDtypeStruct((N, 64), jnp.bfloat16),
        jax.ShapeDtypeStruct((NW, 64), jnp.float32),
    ],
    mesh=plsc.VectorSubcoreMesh(core_axis_name="c", subcore_axis_name="s"),
    compiler_params=pltpu.CompilerParams(
        use_tc_tiling_on_sc=False, needs_layout_passes=False),
    scratch_types=[
        pltpu.VMEM((2, BLK), jnp.float32),
        pltpu.VMEM((32, BLK), jnp.float32),
        pltpu.VMEM((BLK, 64), jnp.float32),
        pltpu.VMEM((BLK, 64), jnp.bfloat16),
        pltpu.VMEM((64,), jnp.float32),
        pltpu.SemaphoreType.DMA,
    ],
)
def _sc_kron(xt_hbm, vt_hbm, kb_hbm, t2_hbm,
             xb_v, vb_v, ko_v, kb_v, t2s_v, ksem):
    wid = lax.axis_index("s") * NC + lax.axis_index("c")
    iota = lax.iota(jnp.int32, 16)
    zero16 = jnp.zeros((16,), jnp.float32)
    nb = jnp.where(wid < (NBLKS % NW), NBLKS // NW + 1, NBLKS // NW)

    def blk_body(j, acc):
        b = wid + j * NW
        gbase = b * BLK

        @pl.when(j > 0)
        def _():
            pltpu.make_async_copy(kb_v, kb_hbm.at[pl.ds(0, BLK)], ksem).wait()

        pltpu.sync_copy(xt_hbm.at[:, pl.ds(gbase, BLK)], xb_v)
        pltpu.sync_copy(vt_hbm.at[:, pl.ds(gbase, BLK)], vb_v)
        for g in range(BLK // 16):
            rows = g * 16 + iota
            gsl = pl.ds(g * 16, 16)
            x0v = xb_v[0, gsl]
            x1v = xb_v[1, gsl]
            for f in range(32):
                vf = vb_v[f, gsl]
                plsc.store_scatter(
                    ko_v, [rows, jnp.full((16,), f, jnp.int32)], x0v * vf)
                plsc.store_scatter(
                    ko_v, [rows, jnp.full((16,), 32 + f, jnp.int32)], x1v * vf)

        def row_body(nn, acc2):
            a0, a1, a2, a3 = acc2
            p0 = ko_v[nn, pl.ds(0, 16)]
            p1 = ko_v[nn, pl.ds(16, 16)]
            p2 = ko_v[nn, pl.ds(32, 16)]
            p3 = ko_v[nn, pl.ds(48, 16)]
            kb_v[nn, pl.ds(0, 32)] = plsc.pack(
                p0, p1, format=plsc.PackFormat.INTERLEAVED)
            kb_v[nn, pl.ds(32, 32)] = plsc.pack(
                p2, p3, format=plsc.PackFormat.INTERLEAVED)
            return (a0 + p0, a1 + p1, a2 + p2, a3 + p3)

        acc = lax.fori_loop(0, BLK, row_body, acc)
        pltpu.async_copy(kb_v, kb_hbm.at[pl.ds(gbase, BLK)], ksem)
        return acc

    acc = lax.fori_loop(0, nb, blk_body, (zero16, zero16, zero16, zero16))
    pltpu.make_async_copy(kb_v, kb_hbm.at[pl.ds(0, BLK)], ksem).wait()
    for c in range(4):
        t2s_v[pl.ds(c * 16, 16)] = acc[c]
    pltpu.sync_copy(t2s_v, t2_hbm.at[wid])


@functools.partial(
    pl.kernel,
    out_type=jax.ShapeDtypeStruct((N, 64), jnp.float32),
    mesh=plsc.VectorSubcoreMesh(core_axis_name="c", subcore_axis_name="s"),
    compiler_params=pltpu.CompilerParams(
        use_tc_tiling_on_sc=False, needs_layout_passes=False),
    scratch_types=[
        pltpu.VMEM((2, NBR, BLK), jnp.int32),
        pltpu.VMEM((2, BLK, 64), jnp.bfloat16),
        pltpu.VMEM((BLK, 64), jnp.bfloat16),
        pltpu.VMEM((BLK, 64), jnp.float32),
        pltpu.VMEM((64,), jnp.float32),
        pltpu.VMEM((64,), jnp.float32),
        pltpu.VMEM((64,), jnp.float32),
        pltpu.SemaphoreType.DMA,
        pltpu.SemaphoreType.DMA,
        pltpu.SemaphoreType.DMA,
    ],
)
def _sc_gather(kb_hbm, nbr_hbm, th1_hbm, th3_hbm, t2_hbm, out_hbm,
               idx_v, s_v, kl_v, o_v, th1_v, th3_v, t2_v,
               sem0, sem1, klsem):
    wid = lax.axis_index("s") * NC + lax.axis_index("c")
    sems = [sem0, sem1]
    pltpu.sync_copy(th1_hbm, th1_v)
    pltpu.sync_copy(th3_hbm, th3_v)
    pltpu.sync_copy(t2_hbm, t2_v)

    zero16 = jnp.zeros((16,), jnp.float32)
    zero32b = jnp.zeros((32,), jnp.bfloat16)

    def zero_body(r, carry):
        for p in range(2):
            for c in range(2):
                s_v[p, r, pl.ds(c * 32, 32)] = zero32b
        return carry

    lax.fori_loop(0, BLK, zero_body, 0)

    th1c = [th1_v[pl.ds(c * 16, 16)] for c in range(4)]
    th3c = [th3_v[pl.ds(c * 16, 16)] for c in range(4)]
    t2c = [t2_v[pl.ds(c * 16, 16)] for c in range(4)]

    def stage(b, p):
        gbase = b * BLK
        pltpu.sync_copy(nbr_hbm.at[:, pl.ds(gbase, BLK)], idx_v.at[p])
        for k in range(NBR):
            pltpu.async_copy(
                kb_hbm.at[idx_v.at[p, k]], s_v.at[p],
                sems[p], add=True)

    def finish(b, p):
        gbase = b * BLK
        cp = pltpu.async_copy(kb_hbm.at[pl.ds(gbase, BLK)], kl_v, klsem)
        for _ in range(NBR):
            pltpu.make_async_copy(
                kb_hbm.at[idx_v.at[p, 0]], s_v.at[p],
                sems[p]).wait()
        cp.wait()

        himask = jnp.full((16,), -65536, jnp.int32)

        def row_body(r, carry):
            for c in range(2):
                sw = plsc.bitcast(s_v[p, r, pl.ds(c * 32, 32)], jnp.int32)
                kw = plsc.bitcast(kl_v[r, pl.ds(c * 32, 32)], jnp.int32)
                s_lo = plsc.bitcast(lax.shift_left(sw, 16), jnp.float32)
                s_hi = plsc.bitcast(sw & himask, jnp.float32)
                k_lo = plsc.bitcast(lax.shift_left(kw, 16), jnp.float32)
                k_hi = plsc.bitcast(kw & himask, jnp.float32)
                for h, (s, kk) in ((0, (s_lo, k_lo)), (1, (s_hi, k_hi))):
                    cc = c * 2 + h
                    sl = pl.ds(cc * 16, 16)
                    o_v[r, sl] = th1c[cc] * kk + th3c[cc] * s + t2c[cc]
                s_v[p, r, pl.ds(c * 32, 32)] = zero32b
            return carry

        lax.fori_loop(0, BLK, row_body, 0)
        pltpu.sync_copy(o_v, out_hbm.at[pl.ds(gbase, BLK)])

    nsteps = (NBLKS + NW - 1) // NW
    b0 = wid
    stage(b0, 0)

    def pair_body(jj, carry):
        b_even = wid + (2 * jj) * NW
        b_odd = b_even + NW
        b_next = b_odd + NW

        @pl.when(b_odd < NBLKS)
        def _():
            stage(b_odd, 1)

        finish(b_even, 0)

        @pl.when(b_next < NBLKS)
        def _():
            stage(b_next, 0)

        @pl.when(b_odd < NBLKS)
        def _():
            finish(b_odd, 1)

        return carry

    lax.fori_loop(0, (nsteps + 1) // 2, pair_body, 0)


def kernel(x, x_v, neighbors_indices, theta1, theta2, theta3):
    n = x.shape[0]
    xt = x.reshape(n, 2).astype(jnp.float32).T
    vt = x_v.reshape(n, 32).astype(jnp.float32).T
    nbrt = neighbors_indices.astype(jnp.int32).T

    kb_table, t2part = _sc_kron(xt, vt)

    th1v = jnp.tile(theta1.astype(jnp.float32), 8)
    th3v = jnp.tile(theta3.astype(jnp.float32), 8) / NBR
    t2v = jnp.tile(theta2.astype(jnp.float32), 8) * jnp.sum(t2part, axis=0) / n

    out = _sc_gather(kb_table, nbrt, th1v, th3v, t2v)
    return out.reshape(n, 8, 8)


# speedup vs baseline: 1.2410x; 1.2410x over previous
import functools

import jax
import jax.numpy as jnp
from jax import lax
from jax.experimental import pallas as pl
from jax.experimental.pallas import tpu as pltpu
from jax.experimental.pallas import tpu_sc as plsc

N = 50000
NBR = 16

NC = 2
NS = 16
NW = NC * NS

BLK = 80
NBLKS = N // BLK

RB = 2000


@functools.partial(
    pl.kernel,
    out_type=[
        jax.ShapeDtypeStruct((N, 64), jnp.bfloat16),
        jax.ShapeDtypeStruct((NW, 64), jnp.float32),
    ],
    mesh=plsc.VectorSubcoreMesh(core_axis_name="c", subcore_axis_name="s"),
    compiler_params=pltpu.CompilerParams(
        use_tc_tiling_on_sc=False, needs_layout_passes=False),
    scratch_types=[
        pltpu.VMEM((2, BLK), jnp.float32),
        pltpu.VMEM((32, BLK), jnp.float32),
        pltpu.VMEM((BLK, 65), jnp.float32),
        pltpu.VMEM((BLK, 64), jnp.bfloat16),
        pltpu.VMEM((64,), jnp.float32),
        pltpu.SemaphoreType.DMA,
    ],
)
def _sc_kron(xt_hbm, vt_hbm, kb_hbm, t2_hbm,
             xb_v, vb_v, ko_v, kb_v, t2s_v, ksem):
    wid = lax.axis_index("s") * NC + lax.axis_index("c")
    iota = lax.iota(jnp.int32, 16)
    zero16 = jnp.zeros((16,), jnp.float32)
    nb = jnp.where(wid < (NBLKS % NW), NBLKS // NW + 1, NBLKS // NW)

    def blk_body(j, acc):
        b = wid + j * NW
        gbase = b * BLK

        @pl.when(j > 0)
        def _():
            pltpu.make_async_copy(kb_v, kb_hbm.at[pl.ds(0, BLK)], ksem).wait()

        pltpu.sync_copy(xt_hbm.at[:, pl.ds(gbase, BLK)], xb_v)
        pltpu.sync_copy(vt_hbm.at[:, pl.ds(gbase, BLK)], vb_v)
        for g in range(BLK // 16):
            rows = g * 16 + iota
            gsl = pl.ds(g * 16, 16)
            x0v = xb_v[0, gsl]
            x1v = xb_v[1, gsl]
            for f in range(32):
                vf = vb_v[f, gsl]
                plsc.store_scatter(
                    ko_v, [rows, jnp.full((16,), f, jnp.int32)], x0v * vf)
                plsc.store_scatter(
                    ko_v, [rows, jnp.full((16,), 32 + f, jnp.int32)], x1v * vf)

        def row_body(nn, acc2):
            a0, a1, a2, a3 = acc2
            p0 = ko_v[nn, pl.ds(0, 16)]
            p1 = ko_v[nn, pl.ds(16, 16)]
            p2 = ko_v[nn, pl.ds(32, 16)]
            p3 = ko_v[nn, pl.ds(48, 16)]
            kb_v[nn, pl.ds(0, 32)] = plsc.pack(
                p0, p1, format=plsc.PackFormat.INTERLEAVED)
            kb_v[nn, pl.ds(32, 32)] = plsc.pack(
                p2, p3, format=plsc.PackFormat.INTERLEAVED)
            return (a0 + p0, a1 + p1, a2 + p2, a3 + p3)

        acc = lax.fori_loop(0, BLK, row_body, acc)
        pltpu.async_copy(kb_v, kb_hbm.at[pl.ds(gbase, BLK)], ksem)
        return acc

    acc = lax.fori_loop(0, nb, blk_body, (zero16, zero16, zero16, zero16))
    pltpu.make_async_copy(kb_v, kb_hbm.at[pl.ds(0, BLK)], ksem).wait()
    for c in range(4):
        t2s_v[pl.ds(c * 16, 16)] = acc[c]
    pltpu.sync_copy(t2s_v, t2_hbm.at[wid])


@functools.partial(
    pl.kernel,
    out_type=jax.ShapeDtypeStruct((N, 64), jnp.float32),
    mesh=plsc.VectorSubcoreMesh(core_axis_name="c", subcore_axis_name="s"),
    compiler_params=pltpu.CompilerParams(
        use_tc_tiling_on_sc=False, needs_layout_passes=False),
    scratch_types=[
        pltpu.VMEM((2, NBR, BLK), jnp.int32),
        pltpu.VMEM((2, BLK, 64), jnp.bfloat16),
        pltpu.VMEM((BLK, 64), jnp.bfloat16),
        pltpu.VMEM((BLK, 64), jnp.float32),
        pltpu.VMEM((64,), jnp.float32),
        pltpu.VMEM((64,), jnp.float32),
        pltpu.VMEM((64,), jnp.float32),
        pltpu.SemaphoreType.DMA,
        pltpu.SemaphoreType.DMA,
        pltpu.SemaphoreType.DMA,
    ],
)
def _sc_gather(kb_hbm, nbr_hbm, th1_hbm, th3_hbm, t2_hbm, out_hbm,
               idx_v, s_v, kl_v, o_v, th1_v, th3_v, t2_v,
               sem0, sem1, klsem):
    wid = lax.axis_index("s") * NC + lax.axis_index("c")
    sems = [sem0, sem1]
    pltpu.sync_copy(th1_hbm, th1_v)
    pltpu.sync_copy(th3_hbm, th3_v)
    pltpu.sync_copy(t2_hbm, t2_v)

    zero16 = jnp.zeros((16,), jnp.float32)
    zero32b = jnp.zeros((32,), jnp.bfloat16)

    def zero_body(r, carry):
        for p in range(2):
            for c in range(2):
                s_v[p, r, pl.ds(c * 32, 32)] = zero32b
        return carry

    lax.fori_loop(0, BLK, zero_body, 0)

    th1c = [th1_v[pl.ds(c * 16, 16)] for c in range(4)]
    th3c = [th3_v[pl.ds(c * 16, 16)] for c in range(4)]
    t2c = [t2_v[pl.ds(c * 16, 16)] for c in range(4)]

    def stage(b, p):
        gbase = b * BLK
        pltpu.sync_copy(nbr_hbm.at[:, pl.ds(gbase, BLK)], idx_v.at[p])
        for k in range(NBR):
            pltpu.async_copy(
                kb_hbm.at[idx_v.at[p, k]], s_v.at[p],
                sems[p], add=True)

    def finish(b, p):
        gbase = b * BLK
        cp = pltpu.async_copy(kb_hbm.at[pl.ds(gbase, BLK)], kl_v, klsem)
        for _ in range(NBR):
            pltpu.make_async_copy(
                kb_hbm.at[idx_v.at[p, 0]], s_v.at[p],
                sems[p]).wait()
        cp.wait()

        himask = jnp.full((16,), -65536, jnp.int32)

        def row_body(r, carry):
            for c in range(2):
                sw = plsc.bitcast(s_v[p, r, pl.ds(c * 32, 32)], jnp.int32)
                kw = plsc.bitcast(kl_v[r, pl.ds(c * 32, 32)], jnp.int32)
                s_lo = plsc.bitcast(lax.shift_left(sw, 16), jnp.float32)
                s_hi = plsc.bitcast(sw & himask, jnp.float32)
                k_lo = plsc.bitcast(lax.shift_left(kw, 16), jnp.float32)
                k_hi = plsc.bitcast(kw & himask, jnp.float32)
                for h, (s, kk) in ((0, (s_lo, k_lo)), (1, (s_hi, k_hi))):
                    cc = c * 2 + h
                    sl = pl.ds(cc * 16, 16)
                    o_v[r, sl] = th1c[cc] * kk + th3c[cc] * s + t2c[cc]
                s_v[p, r, pl.ds(c * 32, 32)] = zero32b
            return carry

        lax.fori_loop(0, BLK, row_body, 0)
        pltpu.sync_copy(o_v, out_hbm.at[pl.ds(gbase, BLK)])

    nsteps = (NBLKS + NW - 1) // NW
    b0 = wid
    stage(b0, 0)

    def pair_body(jj, carry):
        b_even = wid + (2 * jj) * NW
        b_odd = b_even + NW
        b_next = b_odd + NW

        @pl.when(b_odd < NBLKS)
        def _():
            stage(b_odd, 1)

        finish(b_even, 0)

        @pl.when(b_next < NBLKS)
        def _():
            stage(b_next, 0)

        @pl.when(b_odd < NBLKS)
        def _():
            finish(b_odd, 1)

        return carry

    lax.fori_loop(0, (nsteps + 1) // 2, pair_body, 0)


def kernel(x, x_v, neighbors_indices, theta1, theta2, theta3):
    n = x.shape[0]
    xt = x.reshape(n, 2).astype(jnp.float32).T
    vt = x_v.reshape(n, 32).astype(jnp.float32).T
    nbrt = neighbors_indices.astype(jnp.int32).T

    kb_table, t2part = _sc_kron(xt, vt)

    th1v = jnp.tile(theta1.astype(jnp.float32), 8)
    th3v = jnp.tile(theta3.astype(jnp.float32), 8) / NBR
    t2v = jnp.tile(theta2.astype(jnp.float32), 8) * jnp.sum(t2part, axis=0) / n

    out = _sc_gather(kb_table, nbrt, th1v, th3v, t2v)
    return out.reshape(n, 8, 8)
